# baseline (speedup 1.0000x reference)
import functools

import jax
import jax.numpy as jnp
from jax import lax
from jax.experimental import pallas as pl
from jax.experimental.pallas import tpu as pltpu
from jax.experimental.pallas import tpu_sc as plsc

_D = 96
_L = 16
_DV = _D // _L
_ROWS = 16 * 1024
_NC = 2
_NS = 16
_NW = _NC * _NS
_RPW = _ROWS // _NW

_mesh = plsc.VectorSubcoreMesh(core_axis_name="c", subcore_axis_name="s")


@functools.partial(
    pl.kernel,
    mesh=_mesh,
    out_type=jax.ShapeDtypeStruct((_ROWS, _D), jnp.float32),
    scratch_types=[
        pltpu.VMEM((_RPW, _D), jnp.float32),
        pltpu.VMEM((1, _D), jnp.float32),
    ],
)
def _encode(patch_hbm, pos_hbm, out_hbm, buf, posv):
    wid = lax.axis_index("s") * _NC + lax.axis_index("c")
    base = wid * _RPW
    pltpu.sync_copy(pos_hbm.at[pl.ds(0, 1)], posv)
    pltpu.sync_copy(patch_hbm.at[pl.ds(base, _RPW)], buf)
    pv = [posv[0, pl.ds(j * _L, _L)] for j in range(_DV)]

    def row(r, carry):
        for j in range(_DV):
            buf[r, pl.ds(j * _L, _L)] += pv[j]
        return carry

    lax.fori_loop(0, _RPW, row, 0)
    pltpu.sync_copy(buf, out_hbm.at[pl.ds(base, _RPW)])


def kernel(patch, pos_table):
    rows = patch.reshape(-1, _D)
    out = _encode(rows, pos_table)
    return out.reshape(patch.shape)
